# initial kernel scaffold (unmeasured)
import jax
import jax.numpy as jnp
from jax import lax
from jax.experimental import pallas as pl
from jax.experimental.pallas import tpu as pltpu

N_DEV = 4
B = 64
D = 1024
H = 2048


def kernel(x, Win0, Wout0, Win1, Wout1, Win2, Wout2):
    def body(x_ref, win0_ref, wout0_ref, win1_ref, wout1_ref, win2_ref,
             wout2_ref, out_ref, xacts, sendbuf, rsbuf,
             ag_send, ag_recv, rs_send, rs_recv):
        my = lax.axis_index("i")
        right = (my + 1) % N_DEV
        left = (my + N_DEV - 1) % N_DEV
        opp = (my + 2) % N_DEV
        peers = (right, left, opp)

        barrier = pltpu.get_barrier_semaphore()
        for p in peers:
            pl.semaphore_signal(barrier, inc=1, device_id=(p,),
                                device_id_type=pl.DeviceIdType.MESH)
        pl.semaphore_wait(barrier, 3)

        def allgather_my_slot():
            rdmas = []
            for k, p in enumerate(peers):
                r = pltpu.make_async_remote_copy(
                    src_ref=xacts.at[my],
                    dst_ref=xacts.at[my],
                    send_sem=ag_send.at[k],
                    recv_sem=ag_recv.at[k],
                    device_id=(p,),
                    device_id_type=pl.DeviceIdType.MESH,
                )
                r.start()
                rdmas.append(r)
            for r in rdmas:
                r.wait()

        def layer(win_ref, wout_ref):
            xa = xacts[...].reshape(N_DEV * B, D)
            h = jnp.dot(xa, win_ref[...].astype(jnp.bfloat16),
                        preferred_element_type=jnp.float32)
            h = jnp.maximum(h, 0.0).astype(jnp.bfloat16)
            return jnp.dot(h, wout_ref[...].astype(jnp.bfloat16),
                           preferred_element_type=jnp.float32)

        def reduce_scatter(partial):
            sendbuf[...] = partial.reshape(N_DEV, B, D).astype(jnp.bfloat16)
            rdmas = []
            for k, p in enumerate(peers):
                r = pltpu.make_async_remote_copy(
                    src_ref=sendbuf.at[p],
                    dst_ref=rsbuf.at[k],
                    send_sem=rs_send.at[k],
                    recv_sem=rs_recv.at[k],
                    device_id=(p,),
                    device_id_type=pl.DeviceIdType.MESH,
                )
                r.start()
                rdmas.append(r)
            own = lax.dynamic_slice_in_dim(partial, my * B, B, axis=0)
            for r in rdmas:
                r.wait()
            return (own
                    + rsbuf[0].astype(jnp.float32)
                    + rsbuf[1].astype(jnp.float32)
                    + rsbuf[2].astype(jnp.float32))

        xacts[my] = x_ref[...].astype(jnp.bfloat16)
        allgather_my_slot()

        for win_ref, wout_ref in ((win0_ref, wout0_ref),
                                  (win1_ref, wout1_ref)):
            acc = reduce_scatter(layer(win_ref, wout_ref))
            xacts[my] = acc.astype(jnp.bfloat16)
            allgather_my_slot()

        out_ref[...] = reduce_scatter(layer(win2_ref, wout2_ref))

    return pl.pallas_call(
        body,
        out_shape=jax.ShapeDtypeStruct((B, D), jnp.float32),
        in_specs=[pl.BlockSpec(memory_space=pltpu.VMEM)] * 7,
        out_specs=pl.BlockSpec(memory_space=pltpu.VMEM),
        scratch_shapes=[
            pltpu.VMEM((N_DEV, B, D), jnp.bfloat16),
            pltpu.VMEM((N_DEV, B, D), jnp.bfloat16),
            pltpu.VMEM((3, B, D), jnp.bfloat16),
            pltpu.SemaphoreType.DMA((3,)),
            pltpu.SemaphoreType.DMA((3,)),
            pltpu.SemaphoreType.DMA((3,)),
            pltpu.SemaphoreType.DMA((3,)),
        ],
        compiler_params=pltpu.CompilerParams(collective_id=0),
    )(x, Win0, Wout0, Win1, Wout1, Win2, Wout2)


# baseline (device time: 60845 ns/iter reference)
import jax
import jax.numpy as jnp
from jax import lax
from jax.experimental import pallas as pl
from jax.experimental.pallas import tpu as pltpu

N_DEV = 4
B = 64
D = 1024
H = 2048


def kernel(x, Win0, Wout0, Win1, Wout1, Win2, Wout2):
    def body(x_ref, win0_ref, wout0_ref, win1_ref, wout1_ref, win2_ref,
             wout2_ref, out_ref, xacts, sendbuf, rsbuf,
             ag_send, ag_recv, rs_send, rs_recv):
        my = lax.axis_index("i")
        right = (my + 1) % N_DEV
        left = (my + N_DEV - 1) % N_DEV
        opp = (my + 2) % N_DEV
        peers = (right, left, opp)

        barrier = pltpu.get_barrier_semaphore()
        for p in peers:
            pl.semaphore_signal(barrier, inc=1, device_id=(p,),
                                device_id_type=pl.DeviceIdType.MESH)
        pl.semaphore_wait(barrier, 3)

        def allgather_my_slot():
            rdmas = []
            for k, p in enumerate(peers):
                r = pltpu.make_async_remote_copy(
                    src_ref=xacts.at[my],
                    dst_ref=xacts.at[my],
                    send_sem=ag_send.at[k],
                    recv_sem=ag_recv.at[k],
                    device_id=(p,),
                    device_id_type=pl.DeviceIdType.MESH,
                )
                r.start()
                rdmas.append(r)
            for r in rdmas:
                r.wait()

        def layer(win_ref, wout_ref):
            xa = xacts[...].reshape(N_DEV * B, D)
            h = jnp.dot(xa, win_ref[...].astype(jnp.bfloat16),
                        preferred_element_type=jnp.float32)
            h = jnp.maximum(h, 0.0).astype(jnp.bfloat16)
            return jnp.dot(h, wout_ref[...].astype(jnp.bfloat16),
                           preferred_element_type=jnp.float32)

        def reduce_scatter(partial):
            sendbuf[...] = partial.reshape(N_DEV, B, D).astype(jnp.bfloat16)
            rdmas = []
            for k, p in enumerate(peers):
                r = pltpu.make_async_remote_copy(
                    src_ref=sendbuf.at[p],
                    dst_ref=rsbuf.at[k],
                    send_sem=rs_send.at[k],
                    recv_sem=rs_recv.at[k],
                    device_id=(p,),
                    device_id_type=pl.DeviceIdType.MESH,
                )
                r.start()
                rdmas.append(r)
            own = sendbuf[my].astype(jnp.float32)
            for r in rdmas:
                r.wait()
            return (own
                    + rsbuf[0].astype(jnp.float32)
                    + rsbuf[1].astype(jnp.float32)
                    + rsbuf[2].astype(jnp.float32))

        xacts[my] = x_ref[...].astype(jnp.bfloat16)
        allgather_my_slot()

        for win_ref, wout_ref in ((win0_ref, wout0_ref),
                                  (win1_ref, wout1_ref)):
            acc = reduce_scatter(layer(win_ref, wout_ref))
            xacts[my] = acc.astype(jnp.bfloat16)
            allgather_my_slot()

        out_ref[...] = reduce_scatter(layer(win2_ref, wout2_ref))

    return pl.pallas_call(
        body,
        out_shape=jax.ShapeDtypeStruct((B, D), jnp.float32),
        in_specs=[pl.BlockSpec(memory_space=pltpu.VMEM)] * 7,
        out_specs=pl.BlockSpec(memory_space=pltpu.VMEM),
        scratch_shapes=[
            pltpu.VMEM((N_DEV, B, D), jnp.bfloat16),
            pltpu.VMEM((N_DEV, B, D), jnp.bfloat16),
            pltpu.VMEM((3, B, D), jnp.bfloat16),
            pltpu.SemaphoreType.DMA((3,)),
            pltpu.SemaphoreType.DMA((3,)),
            pltpu.SemaphoreType.DMA((3,)),
            pltpu.SemaphoreType.DMA((3,)),
        ],
        compiler_params=pltpu.CompilerParams(
            collective_id=0, vmem_limit_bytes=100 * 1024 * 1024),
    )(x, Win0, Wout0, Win1, Wout1, Win2, Wout2)


# device time: 48760 ns/iter; 1.2478x vs baseline; 1.2478x over previous
import jax
import jax.numpy as jnp
from jax import lax
from jax.experimental import pallas as pl
from jax.experimental.pallas import tpu as pltpu

N_DEV = 4
B = 64
D = 1024
H = 2048
BF = jnp.bfloat16
F32 = jnp.float32


def kernel(x, Win0, Wout0, Win1, Wout1, Win2, Wout2):
    def body(x_ref, win0_ref, wout0_ref, win1_ref, wout1_ref, win2_ref,
             wout2_ref, out_ref,
             xacts, sendbuf, rsbuf, winbuf, woutbuf,
             ag_send, ag_recv, rs_send, rs_recv, wsem):
        my = lax.axis_index("i")

        wcopies = []
        for l, (wi, wo) in enumerate(((win0_ref, wout0_ref),
                                      (win1_ref, wout1_ref),
                                      (win2_ref, wout2_ref))):
            ci = pltpu.make_async_copy(wi, winbuf.at[l], wsem.at[2 * l])
            co = pltpu.make_async_copy(wo, woutbuf.at[l], wsem.at[2 * l + 1])
            ci.start()
            co.start()
            wcopies.append((ci, co))

        xacts[my] = x_ref[...].astype(BF)

        barrier = pltpu.get_barrier_semaphore()
        for t in (1, 2, 3):
            pl.semaphore_signal(barrier, inc=1,
                                device_id=((my + t) % N_DEV,),
                                device_id_type=pl.DeviceIdType.MESH)
        pl.semaphore_wait(barrier, 3)

        def start_ag():
            rdmas = []
            for t in (1, 2, 3):
                r = pltpu.make_async_remote_copy(
                    src_ref=xacts.at[my],
                    dst_ref=xacts.at[my],
                    send_sem=ag_send.at[t],
                    recv_sem=ag_recv.at[t],
                    device_id=((my + t) % N_DEV,),
                    device_id_type=pl.DeviceIdType.MESH,
                )
                r.start()
                rdmas.append(r)
            return rdmas

        def wait_ag_chunk(t):
            p = (my + t) % N_DEV
            pltpu.make_async_remote_copy(
                src_ref=xacts.at[my],
                dst_ref=xacts.at[p],
                send_sem=ag_send.at[t],
                recv_sem=ag_recv.at[4 - t],
                device_id=(p,),
                device_id_type=pl.DeviceIdType.MESH,
            ).wait_recv()

        def layer(l):
            wcopies[l][0].wait()
            win_bf = winbuf[l].astype(BF)
            h_m = jnp.maximum(
                jnp.dot(xacts[my], win_bf, preferred_element_type=F32),
                0.0).astype(BF)
            wcopies[l][1].wait()
            wout_bf = woutbuf[l].astype(BF)
            own = jnp.dot(h_m, wout_bf, preferred_element_type=F32)

            rs_sends = []
            for t in (1, 2, 3):
                p = (my + t) % N_DEV
                wait_ag_chunk(t)
                h_p = jnp.maximum(
                    jnp.dot(xacts[p], win_bf, preferred_element_type=F32),
                    0.0).astype(BF)
                part = jnp.dot(h_p, wout_bf, preferred_element_type=F32)
                sendbuf[p] = part.astype(BF)
                r = pltpu.make_async_remote_copy(
                    src_ref=sendbuf.at[p],
                    dst_ref=rsbuf.at[t],
                    send_sem=rs_send.at[t],
                    recv_sem=rs_recv.at[t],
                    device_id=(p,),
                    device_id_type=pl.DeviceIdType.MESH,
                )
                r.start()
                rs_sends.append(r)

            for t in (1, 2, 3):
                p = (my + t) % N_DEV
                pltpu.make_async_remote_copy(
                    src_ref=sendbuf.at[p],
                    dst_ref=rsbuf.at[4 - t],
                    send_sem=rs_send.at[t],
                    recv_sem=rs_recv.at[4 - t],
                    device_id=(p,),
                    device_id_type=pl.DeviceIdType.MESH,
                ).wait_recv()
            acc = (own + rsbuf[1].astype(F32) + rsbuf[2].astype(F32)
                   + rsbuf[3].astype(F32))
            for r in rs_sends:
                r.wait_send()
            return acc

        ag_sends = start_ag()
        for l in (0, 1):
            acc = layer(l)
            for r in ag_sends:
                r.wait_send()
            xacts[my] = acc.astype(BF)
            ag_sends = start_ag()
        acc = layer(2)
        for r in ag_sends:
            r.wait_send()
        out_ref[...] = acc

    return pl.pallas_call(
        body,
        out_shape=jax.ShapeDtypeStruct((B, D), jnp.float32),
        in_specs=[pl.BlockSpec(memory_space=pltpu.VMEM)]
        + [pl.BlockSpec(memory_space=pl.ANY)] * 6,
        out_specs=pl.BlockSpec(memory_space=pltpu.VMEM),
        scratch_shapes=[
            pltpu.VMEM((N_DEV, B, D), BF),
            pltpu.VMEM((N_DEV, B, D), BF),
            pltpu.VMEM((N_DEV, B, D), BF),
            pltpu.VMEM((3, D, H), F32),
            pltpu.VMEM((3, H, D), F32),
            pltpu.SemaphoreType.DMA((N_DEV,)),
            pltpu.SemaphoreType.DMA((N_DEV,)),
            pltpu.SemaphoreType.DMA((N_DEV,)),
            pltpu.SemaphoreType.DMA((N_DEV,)),
            pltpu.SemaphoreType.DMA((6,)),
        ],
        compiler_params=pltpu.CompilerParams(
            collective_id=0, vmem_limit_bytes=100 * 1024 * 1024),
    )(x, Win0, Wout0, Win1, Wout1, Win2, Wout2)


# device time: 44422 ns/iter; 1.3697x vs baseline; 1.0977x over previous
import jax
import jax.numpy as jnp
from jax import lax
from jax.experimental import pallas as pl
from jax.experimental.pallas import tpu as pltpu

N_DEV = 4
B = 64
D = 1024
H = 2048
BF = jnp.bfloat16
F32 = jnp.float32


def kernel(x, Win0, Wout0, Win1, Wout1, Win2, Wout2):
    def body(x_ref, win0_ref, wout0_ref, win1_ref, wout1_ref, win2_ref,
             wout2_ref, out_ref,
             xacts, sendbuf, rsbuf, winbuf, woutbuf, winb, woutb,
             ag_send, ag_recv, rs_send, rs_recv, wsem):
        my = lax.axis_index("i")

        w_refs = ((win0_ref, wout0_ref), (win1_ref, wout1_ref),
                  (win2_ref, wout2_ref))

        def start_wcopy(l):
            ci = pltpu.make_async_copy(w_refs[l][0], winbuf.at[l % 2],
                                       wsem.at[2 * l])
            co = pltpu.make_async_copy(w_refs[l][1], woutbuf.at[l % 2],
                                       wsem.at[2 * l + 1])
            ci.start()
            co.start()
            return (ci, co)

        wcopies = [start_wcopy(0), start_wcopy(1), None]

        xacts[my] = x_ref[...].astype(BF)

        barrier = pltpu.get_barrier_semaphore()
        for t in (1, 2, 3):
            pl.semaphore_signal(barrier, inc=1,
                                device_id=((my + t) % N_DEV,),
                                device_id_type=pl.DeviceIdType.MESH)
        pl.semaphore_wait(barrier, 3)

        def start_ag():
            rdmas = []
            for t in (1, 2, 3):
                r = pltpu.make_async_remote_copy(
                    src_ref=xacts.at[my],
                    dst_ref=xacts.at[my],
                    send_sem=ag_send.at[t],
                    recv_sem=ag_recv.at[t],
                    device_id=((my + t) % N_DEV,),
                    device_id_type=pl.DeviceIdType.MESH,
                )
                r.start()
                rdmas.append(r)
            return rdmas

        def wait_ag_chunks():
            for t in (1, 2, 3):
                p = (my + t) % N_DEV
                pltpu.make_async_remote_copy(
                    src_ref=xacts.at[my],
                    dst_ref=xacts.at[p],
                    send_sem=ag_send.at[t],
                    recv_sem=ag_recv.at[4 - t],
                    device_id=(p,),
                    device_id_type=pl.DeviceIdType.MESH,
                ).wait_recv()

        def layer(l):
            wcopies[l][0].wait()
            winb[...] = winbuf[l % 2].astype(BF)
            wcopies[l][1].wait()
            woutb[...] = woutbuf[l % 2].astype(BF)
            if l == 0:
                wcopies[2] = start_wcopy(2)

            wait_ag_chunks()
            xa = xacts[...].reshape(N_DEV * B, D)
            h = jnp.maximum(jnp.dot(xa, winb[...],
                                    preferred_element_type=F32),
                            0.0).astype(BF)
            part = jnp.dot(h, woutb[...], preferred_element_type=F32)
            sendbuf[...] = part.reshape(N_DEV, B, D).astype(BF)

            rs_sends = []
            for t in (1, 2, 3):
                p = (my + t) % N_DEV
                r = pltpu.make_async_remote_copy(
                    src_ref=sendbuf.at[p],
                    dst_ref=rsbuf.at[t],
                    send_sem=rs_send.at[t],
                    recv_sem=rs_recv.at[t],
                    device_id=(p,),
                    device_id_type=pl.DeviceIdType.MESH,
                )
                r.start()
                rs_sends.append(r)

            for t in (1, 2, 3):
                p = (my + t) % N_DEV
                pltpu.make_async_remote_copy(
                    src_ref=sendbuf.at[p],
                    dst_ref=rsbuf.at[4 - t],
                    send_sem=rs_send.at[t],
                    recv_sem=rs_recv.at[4 - t],
                    device_id=(p,),
                    device_id_type=pl.DeviceIdType.MESH,
                ).wait_recv()
            acc = (sendbuf[my].astype(F32) + rsbuf[1].astype(F32)
                   + rsbuf[2].astype(F32) + rsbuf[3].astype(F32))
            for r in rs_sends:
                r.wait_send()
            return acc

        ag_sends = start_ag()
        for l in (0, 1):
            acc = layer(l)
            for r in ag_sends:
                r.wait_send()
            xacts[my] = acc.astype(BF)
            ag_sends = start_ag()
        acc = layer(2)
        for r in ag_sends:
            r.wait_send()
        out_ref[...] = acc

    return pl.pallas_call(
        body,
        out_shape=jax.ShapeDtypeStruct((B, D), jnp.float32),
        in_specs=[pl.BlockSpec(memory_space=pltpu.VMEM)]
        + [pl.BlockSpec(memory_space=pl.ANY)] * 6,
        out_specs=pl.BlockSpec(memory_space=pltpu.VMEM),
        scratch_shapes=[
            pltpu.VMEM((N_DEV, B, D), BF),
            pltpu.VMEM((N_DEV, B, D), BF),
            pltpu.VMEM((N_DEV, B, D), BF),
            pltpu.VMEM((2, D, H), F32),
            pltpu.VMEM((2, H, D), F32),
            pltpu.VMEM((D, H), BF),
            pltpu.VMEM((H, D), BF),
            pltpu.SemaphoreType.DMA((N_DEV,)),
            pltpu.SemaphoreType.DMA((N_DEV,)),
            pltpu.SemaphoreType.DMA((N_DEV,)),
            pltpu.SemaphoreType.DMA((N_DEV,)),
            pltpu.SemaphoreType.DMA((6,)),
        ],
        compiler_params=pltpu.CompilerParams(
            collective_id=0, vmem_limit_bytes=100 * 1024 * 1024),
    )(x, Win0, Wout0, Win1, Wout1, Win2, Wout2)
